# baseline (speedup 1.0000x reference)
import functools

import jax
import jax.numpy as jnp
from jax import lax
from jax.experimental import pallas as pl
from jax.experimental.pallas import tpu as pltpu
from jax.experimental.pallas import tpu_sc as plsc

NUM_CORES = 2
NUM_SUBCORES = 16
NUM_WORKERS = NUM_CORES * NUM_SUBCORES
IDX_CHUNK = 128


def _sc_gather(table, idx2d, batch, dim):
    rows_per_w = batch // NUM_WORKERS
    chunks_per_w = rows_per_w // IDX_CHUNK

    mesh = plsc.VectorSubcoreMesh(core_axis_name="c", subcore_axis_name="s")

    @functools.partial(
        pl.kernel,
        mesh=mesh,
        out_type=jax.ShapeDtypeStruct((batch, dim), jnp.float32),
        scratch_types=[
            pltpu.VMEM((chunks_per_w, IDX_CHUNK), jnp.int32),
            pltpu.VMEM((rows_per_w, dim), jnp.float32),
        ]
        + [pltpu.SemaphoreType.DMA] * chunks_per_w
        + [pltpu.SemaphoreType.DMA],
    )
    def gather_kernel(table_hbm, idx_hbm, out_hbm, idx_v, rows_v, *sems):
        gsems, wsem = sems[:chunks_per_w], sems[chunks_per_w]
        wid = lax.axis_index("s") * NUM_CORES + lax.axis_index("c")
        row_base = wid * rows_per_w
        chunk_base = wid * chunks_per_w
        pltpu.sync_copy(idx_hbm.at[pl.ds(chunk_base, chunks_per_w)], idx_v)
        gathers = []
        for j in range(chunks_per_w):
            gathers.append(
                pltpu.async_copy(
                    table_hbm.at[idx_v.at[j]],
                    rows_v.at[pl.ds(j * IDX_CHUNK, IDX_CHUNK)],
                    gsems[j],
                )
            )
        writes = []
        for j in range(chunks_per_w):
            gathers[j].wait()
            writes.append(
                pltpu.async_copy(
                    rows_v.at[pl.ds(j * IDX_CHUNK, IDX_CHUNK)],
                    out_hbm.at[pl.ds(row_base + j * IDX_CHUNK, IDX_CHUNK)],
                    wsem,
                )
            )
        for w in writes:
            w.wait()

    return gather_kernel(table, idx2d)


def _tc_mlp(x, W1, b1, W2, b2, block):
    batch, dim = x.shape

    def body(x_ref, w1_ref, b1_ref, w2_ref, b2_ref, o_ref):
        x = x_ref[...].astype(jnp.bfloat16)
        h = lax.dot_general(
            x, w1_ref[...].astype(jnp.bfloat16),
            dimension_numbers=(((1,), (1,)), ((), ())),
            preferred_element_type=jnp.float32,
        )
        h = jnp.maximum(h + b1_ref[...], 0.0).astype(jnp.bfloat16)
        o = lax.dot_general(
            h, w2_ref[...].astype(jnp.bfloat16),
            dimension_numbers=(((1,), (1,)), ((), ())),
            preferred_element_type=jnp.float32,
        )
        o_ref[...] = o + b2_ref[...]

    return pl.pallas_call(
        body,
        grid=(batch // block,),
        in_specs=[
            pl.BlockSpec((block, dim), lambda i: (i, 0)),
            pl.BlockSpec((dim, dim), lambda i: (0, 0)),
            pl.BlockSpec((1, dim), lambda i: (0, 0)),
            pl.BlockSpec((dim, dim), lambda i: (0, 0)),
            pl.BlockSpec((1, dim), lambda i: (0, 0)),
        ],
        out_specs=pl.BlockSpec((block, dim), lambda i: (i, 0)),
        out_shape=jax.ShapeDtypeStruct((batch, dim), jnp.float32),
    )(x, W1, b1.reshape(1, dim), W2, b2.reshape(1, dim))


def kernel(writer_ids, table, W1, b1, W2, b2):
    batch = writer_ids.shape[0]
    dim = table.shape[1]
    idx2d = writer_ids.astype(jnp.int32).reshape(batch // IDX_CHUNK, IDX_CHUNK)
    gathered = _sc_gather(table, idx2d, batch, dim)
    return _tc_mlp(gathered, W1, b1, W2, b2, block=16384)


# speedup vs baseline: 1.0625x; 1.0625x over previous
import functools

import jax
import jax.numpy as jnp
from jax import lax
from jax.experimental import pallas as pl
from jax.experimental.pallas import tpu as pltpu
from jax.experimental.pallas import tpu_sc as plsc

NUM_CORES = 2
NUM_SUBCORES = 16
NUM_WORKERS = NUM_CORES * NUM_SUBCORES
IDX_CHUNK = 128


def _sc_gather(table, idx2d, batch, dim):
    rows_per_w = batch // NUM_WORKERS
    chunks_per_w = rows_per_w // IDX_CHUNK

    mesh = plsc.VectorSubcoreMesh(core_axis_name="c", subcore_axis_name="s")

    @functools.partial(
        pl.kernel,
        mesh=mesh,
        out_type=jax.ShapeDtypeStruct((batch, dim), jnp.float32),
        scratch_types=[
            pltpu.VMEM((chunks_per_w, IDX_CHUNK), jnp.int32),
            pltpu.VMEM((rows_per_w, dim), jnp.float32),
        ]
        + [pltpu.SemaphoreType.DMA] * chunks_per_w
        + [pltpu.SemaphoreType.DMA],
    )
    def gather_kernel(table_hbm, idx_hbm, out_hbm, idx_v, rows_v, *sems):
        gsems, wsem = sems[:chunks_per_w], sems[chunks_per_w]
        wid = lax.axis_index("s") * NUM_CORES + lax.axis_index("c")
        row_base = wid * rows_per_w
        chunk_base = wid * chunks_per_w
        pltpu.sync_copy(idx_hbm.at[pl.ds(chunk_base, chunks_per_w)], idx_v)
        gathers = []
        for j in range(chunks_per_w):
            gathers.append(
                pltpu.async_copy(
                    table_hbm.at[idx_v.at[j]],
                    rows_v.at[pl.ds(j * IDX_CHUNK, IDX_CHUNK)],
                    gsems[j],
                )
            )
        writes = []
        for j in range(chunks_per_w):
            gathers[j].wait()
            writes.append(
                pltpu.async_copy(
                    rows_v.at[pl.ds(j * IDX_CHUNK, IDX_CHUNK)],
                    out_hbm.at[pl.ds(row_base + j * IDX_CHUNK, IDX_CHUNK)],
                    wsem,
                )
            )
        for w in writes:
            w.wait()

    return gather_kernel(table, idx2d)


def _tc_mlp(x, W1, b1, W2, b2, block):
    batch, dim = x.shape

    def body(x_ref, w1_ref, b1_ref, w2_ref, b2_ref, o_ref):
        x = x_ref[...].astype(jnp.bfloat16)
        h = lax.dot_general(
            x, w1_ref[...].astype(jnp.bfloat16),
            dimension_numbers=(((1,), (1,)), ((), ())),
            preferred_element_type=jnp.float32,
        )
        h = jnp.maximum(h + b1_ref[...], 0.0).astype(jnp.bfloat16)
        o = lax.dot_general(
            h, w2_ref[...].astype(jnp.bfloat16),
            dimension_numbers=(((1,), (1,)), ((), ())),
            preferred_element_type=jnp.float32,
        )
        o_ref[...] = o + b2_ref[...]

    return pl.pallas_call(
        body,
        grid=(batch // block,),
        in_specs=[
            pl.BlockSpec((block, dim), lambda i: (i, 0)),
            pl.BlockSpec((dim, dim), lambda i: (0, 0)),
            pl.BlockSpec((1, dim), lambda i: (0, 0)),
            pl.BlockSpec((dim, dim), lambda i: (0, 0)),
            pl.BlockSpec((1, dim), lambda i: (0, 0)),
        ],
        out_specs=pl.BlockSpec((block, dim), lambda i: (i, 0)),
        out_shape=jax.ShapeDtypeStruct((batch, dim), jnp.float32),
    )(x, W1, b1.reshape(1, dim), W2, b2.reshape(1, dim))


def kernel(writer_ids, table, W1, b1, W2, b2):
    batch = writer_ids.shape[0]
    dim = table.shape[1]
    idx2d = writer_ids.astype(jnp.int32).reshape(batch // IDX_CHUNK, IDX_CHUNK)
    gathered = _sc_gather(table, idx2d, batch, dim)
    return _tc_mlp(gathered, W1, b1, W2, b2, block=8192)
